# baseline (speedup 1.0000x reference)
import jax
import jax.numpy as jnp
from jax import lax
from jax.experimental import pallas as pl
from jax.experimental.pallas import tpu as pltpu
from jax.experimental.pallas import tpu_sc as plsc

N = 10000
E = 320000
D = 128
L = 16
NS = 16
EDGES_PER_TILE = E // NS
K = 112
NCHUNK = 180
EPAD = NCHUNK * K
NBUF = 3
GLA = 2
SLA = 1
NIB = 4
UNROLL = 12
RCHUNK = 80
NRCHUNK = N // RCHUNK
RPASS = (NRCHUNK + NS - 1) // NS

MM_BLOCK = 400


def _mm_body(x_ref, wln_ref, bln_ref, w1_ref, w2_ref, x0_ref, h1_ref, h2_ref):
    xb = x_ref[...]
    x0_ref[...] = (
        jnp.dot(xb, wln_ref[...], preferred_element_type=jnp.float32)
        + bln_ref[...]
    )
    h1_ref[...] = jnp.dot(xb, w1_ref[...], preferred_element_type=jnp.float32)
    h2_ref[...] = jnp.dot(xb, w2_ref[...], preferred_element_type=jnp.float32)


@jax.jit
def _matmuls(x, W_ln, b_ln, W1, W2):
    grid = (N // MM_BLOCK,)
    blk = pl.BlockSpec((MM_BLOCK, D), lambda i: (i, 0))
    wspec = pl.BlockSpec((D, D), lambda i: (0, 0))
    bspec = pl.BlockSpec((1, D), lambda i: (0, 0))
    return pl.pallas_call(
        _mm_body,
        grid=grid,
        in_specs=[blk, wspec, bspec, wspec, wspec],
        out_specs=[blk, blk, blk],
        out_shape=[jax.ShapeDtypeStruct((N, D), jnp.float32)] * 3,
    )(x, W_ln, b_ln.reshape(1, D), W1, W2)


def _pack_idx(src, dst):
    def pad(v):
        v = v.reshape(NS, EDGES_PER_TILE)
        v = jnp.pad(v, ((0, 0), (0, EPAD - EDGES_PER_TILE)))
        return v.reshape(NS, NCHUNK, 1, K)

    return jnp.concatenate([pad(src), pad(dst)], axis=2)


def _pad_w(wgt):
    w = wgt.reshape(NS, EDGES_PER_TILE)
    w = jnp.pad(w, ((0, 0), (0, EPAD - EDGES_PER_TILE)))
    return w.reshape(NS, NCHUNK, K)


def _process_branch(s, h_hbm, idx_hbm, w_hbm, acc,
                    ibufs, wbufs, rows, isems, sems):
    def istart(c, i):
        pltpu.async_copy(idx_hbm.at[s, c], ibufs[i], isems[i])
        pltpu.async_copy(w_hbm.at[s, c], wbufs[i], isems[i])

    def iwait(c, i):
        pltpu.make_async_copy(idx_hbm.at[s, c], ibufs[i], isems[i]).wait()
        pltpu.make_async_copy(w_hbm.at[s, c], wbufs[i], isems[i]).wait()

    for i in range(NIB - 1):
        istart(i, i)

    for b in range(GLA):
        iwait(b, b)
        pltpu.async_copy(h_hbm.at[ibufs[b].at[0]], rows[b], sems[b])

    def mult(wb, r):
        def grp_body(g16, _):
            w16 = wb[pl.ds(g16 * L, L)]
            for i in range(L):
                e = g16 * L + i
                ws = w16[i]
                for j in range(D // L):
                    sl = pl.ds(j * L, L)
                    r[e, sl] = r[e, sl] * ws
            return 0

        lax.fori_loop(0, K // L, grp_body, 0)

    def outer(gg, _):
        for u in range(UNROLL):
            g = gg * UNROLL + u
            b = u % NBUF
            bd = (u - SLA) % NBUF
            ib = u % NIB
            ibd = (u - SLA) % NIB

            pltpu.make_async_copy(
                h_hbm.at[ibufs[ib].at[0]], rows[b], sems[b]).wait()
            mult(wbufs[ib], rows[b])

            @pl.when(g >= SLA)
            def _():
                pltpu.make_async_copy(
                    rows[bd], acc.at[ibufs[ibd].at[1]], sems[bd]).wait()

            @pl.when(g + NIB - 1 < NCHUNK)
            def _():
                istart(g + NIB - 1, ibd)

            @pl.when(g + GLA < NCHUNK)
            def _():
                ig = (u + GLA) % NIB
                bg = (u + GLA) % NBUF
                iwait(g + GLA, ig)
                pltpu.async_copy(
                    h_hbm.at[ibufs[ig].at[0]], rows[bg], sems[bg])

            pltpu.async_copy(
                rows[b], acc.at[ibufs[ib].at[1]], sems[b], add=True)
        return 0

    lax.fori_loop(0, NCHUNK // UNROLL, outer, 0)

    for g in range(NCHUNK - SLA, NCHUNK):
        pltpu.make_async_copy(
            rows[g % NBUF], acc.at[ibufs[g % NIB].at[1]],
            sems[g % NBUF]).wait()


def _emit_out(s, acc, b_hbm, out_hbm, bv, stage):
    pltpu.sync_copy(b_hbm, bv)
    bvals = [bv[pl.ds(j * L, L)] for j in range(D // L)]

    for i in range(RPASS):
        cid = s + i * NS

        @pl.when(cid < NRCHUNK)
        def _():
            base = cid * RCHUNK
            pltpu.sync_copy(acc.at[pl.ds(base, RCHUNK)],
                            stage.at[pl.ds(0, RCHUNK)])

            def r_body(r, _):
                for j in range(D // L):
                    sl = pl.ds(j * L, L)
                    stage[r, sl] = stage[r, sl] + bvals[j]
                return 0

            lax.fori_loop(0, RCHUNK, r_body, 0)
            pltpu.sync_copy(stage.at[pl.ds(0, RCHUNK)],
                            out_hbm.at[pl.ds(base, RCHUNK)])


def _sc_body(h1, idx1, w1, h2, idx2, w2, b1, b2,
             x1_out, x2_out,
             acc, i0, i1, i2, i3,
             w0, w1b, w2b, w3b,
             r0, r1, r2, bv,
             is0, is1, is2, is3,
             sem0, sem1, sem2):
    c = lax.axis_index("c")
    s = lax.axis_index("s")
    ibufs = [i0, i1, i2, i3]
    wbufs = [w0, w1b, w2b, w3b]
    isems = [is0, is1, is2, is3]
    rows = [r0, r1, r2]
    sems = [sem0, sem1, sem2]
    stage = r0

    def z_body(r, _):
        for j in range(D // L):
            stage[r, pl.ds(j * L, L)] = jnp.zeros((L,), jnp.float32)
        return 0

    lax.fori_loop(0, RCHUNK, z_body, 0)
    for i in range(RPASS):
        cid = s + i * NS

        @pl.when(cid < NRCHUNK)
        def _():
            pltpu.sync_copy(stage.at[pl.ds(0, RCHUNK)],
                            acc.at[pl.ds(cid * RCHUNK, RCHUNK)])

    plsc.subcore_barrier()

    @pl.when(c == 0)
    def _():
        _process_branch(s, h1, idx1, w1, acc,
                        ibufs, wbufs, rows, isems, sems)

    @pl.when(c == 1)
    def _():
        _process_branch(s, h2, idx2, w2, acc,
                        ibufs, wbufs, rows, isems, sems)

    plsc.subcore_barrier()

    @pl.when(c == 0)
    def _():
        _emit_out(s, acc, b1, x1_out, bv, stage)

    @pl.when(c == 1)
    def _():
        _emit_out(s, acc, b2, x2_out, bv, stage)


@jax.jit
def _sc_scatter(h1, idx1, w1, h2, idx2, w2, b1, b2):
    mesh = plsc.VectorSubcoreMesh(core_axis_name="c", subcore_axis_name="s")
    return pl.kernel(
        _sc_body,
        out_type=[jax.ShapeDtypeStruct((N, D), jnp.float32)] * 2,
        mesh=mesh,
        scratch_types=(
            [pltpu.VMEM_SHARED((N, D), jnp.float32)]
            + [pltpu.VMEM((2, K), jnp.int32)] * NIB
            + [pltpu.VMEM((K,), jnp.float32)] * NIB
            + [pltpu.VMEM((K, D), jnp.float32)] * NBUF
            + [pltpu.VMEM((D,), jnp.float32)]
            + [pltpu.SemaphoreType.DMA] * (NIB + NBUF)
        ),
    )(h1, idx1, w1, h2, idx2, w2, b1, b2)


def kernel(x, edge_index, edge_weight, edge_index2, edge_weight2,
           W_ln, b_ln, W1, b1, W2, b2):
    x0, h1, h2 = _matmuls(x, W_ln, b_ln, W1, W2)
    x1, x2 = _sc_scatter(
        h1, _pack_idx(edge_index[0], edge_index[1]), _pad_w(edge_weight),
        h2, _pack_idx(edge_index2[0], edge_index2[1]), _pad_w(edge_weight2),
        b1, b2)
    return (x0, x1, x2)


# speedup vs baseline: 1.0969x; 1.0969x over previous
import jax
import jax.numpy as jnp
from jax import lax
from jax.experimental import pallas as pl
from jax.experimental.pallas import tpu as pltpu
from jax.experimental.pallas import tpu_sc as plsc

N = 10000
E = 320000
D = 128
L = 16
NS = 16
K = 128
TOTAL = E // K
NITER = TOTAL // NS
NBUF = 2
NIB = 6
UNROLL = 6
RCHUNK = 80
NRCHUNK = N // RCHUNK
RPASS = (NRCHUNK + NS - 1) // NS

MM_BLOCK = 400


def _mm_body(x_ref, wln_ref, bln_ref, w1_ref, w2_ref, x0_ref, h1_ref, h2_ref):
    xb = x_ref[...]
    x0_ref[...] = (
        jnp.dot(xb, wln_ref[...], preferred_element_type=jnp.float32)
        + bln_ref[...]
    )
    h1_ref[...] = jnp.dot(xb, w1_ref[...], preferred_element_type=jnp.float32)
    h2_ref[...] = jnp.dot(xb, w2_ref[...], preferred_element_type=jnp.float32)


@jax.jit
def _matmuls(x, W_ln, b_ln, W1, W2):
    grid = (N // MM_BLOCK,)
    blk = pl.BlockSpec((MM_BLOCK, D), lambda i: (i, 0))
    wspec = pl.BlockSpec((D, D), lambda i: (0, 0))
    bspec = pl.BlockSpec((1, D), lambda i: (0, 0))
    return pl.pallas_call(
        _mm_body,
        grid=grid,
        in_specs=[blk, wspec, bspec, wspec, wspec],
        out_specs=[blk, blk, blk],
        out_shape=[jax.ShapeDtypeStruct((N, D), jnp.float32)] * 3,
    )(x, W_ln, b_ln.reshape(1, D), W1, W2)


def _process_branch(s, h_hbm, idx_hbm, w_hbm, acc,
                    ibufs, wbufs, rows, isems, sems):
    def cofs(i):
        return (s + NS * i) * K

    def istart(i, u):
        o = cofs(i)
        pltpu.async_copy(idx_hbm.at[pl.ds(o, K)], ibufs[u].at[0], isems[u])
        pltpu.async_copy(idx_hbm.at[pl.ds(E + o, K)], ibufs[u].at[1],
                         isems[u])
        pltpu.async_copy(w_hbm.at[pl.ds(o, K)], wbufs[u], isems[u])

    def iwait(i, u):
        o = cofs(i)
        pltpu.make_async_copy(
            idx_hbm.at[pl.ds(o, K)], ibufs[u].at[0], isems[u]).wait()
        pltpu.make_async_copy(
            idx_hbm.at[pl.ds(E + o, K)], ibufs[u].at[1], isems[u]).wait()
        pltpu.make_async_copy(
            w_hbm.at[pl.ds(o, K)], wbufs[u], isems[u]).wait()

    for i in range(NBUF + 2):
        istart(i, i)

    iwait(0, 0)
    pltpu.async_copy(h_hbm.at[ibufs[0].at[0]], rows[0], sems[0])

    def mult(wb, r):
        def grp_body(g16, _):
            w16 = wb[pl.ds(g16 * L, L)]
            for i in range(L):
                e = g16 * L + i
                ws = w16[i]
                for j in range(D // L):
                    sl = pl.ds(j * L, L)
                    r[e, sl] = r[e, sl] * ws
            return 0

        lax.fori_loop(0, K // L, grp_body, 0)

    def outer(gg, _):
        for u in range(UNROLL):
            i = gg * UNROLL + u
            b = u % NBUF
            bp = (u - 1) % NBUF
            ib = u % NIB
            ibp = (u - 1) % NIB
            ibn = (u + 1) % NIB
            ib4 = (u + 4) % NIB

            pltpu.make_async_copy(
                h_hbm.at[ibufs[ib].at[0]], rows[b], sems[b]).wait()
            mult(wbufs[ib], rows[b])

            @pl.when(i > 0)
            def _():
                pltpu.make_async_copy(
                    rows[bp], acc.at[ibufs[ibp].at[1]], sems[bp]).wait()

            @pl.when(s + NS * (i + 4) < TOTAL)
            def _():
                istart(i + 4, ib4)

            @pl.when(s + NS * (i + 1) < TOTAL)
            def _():
                iwait(i + 1, ibn)
                pltpu.async_copy(
                    h_hbm.at[ibufs[ibn].at[0]], rows[bp], sems[bp])

            pltpu.async_copy(
                rows[b], acc.at[ibufs[ib].at[1]], sems[b], add=True)
        return 0

    lax.fori_loop(0, NITER // UNROLL, outer, 0)

    ul = (NITER - 1) % UNROLL
    pltpu.make_async_copy(
        rows[ul % NBUF], acc.at[ibufs[ul % NIB].at[1]],
        sems[ul % NBUF]).wait()

    @pl.when(s + NS * NITER < TOTAL)
    def _():
        ut = NITER % UNROLL
        pltpu.make_async_copy(
            h_hbm.at[ibufs[ut % NIB].at[0]], rows[ut % NBUF],
            sems[ut % NBUF]).wait()
        mult(wbufs[ut % NIB], rows[ut % NBUF])
        pltpu.sync_copy(rows[ut % NBUF], acc.at[ibufs[ut % NIB].at[1]],
                        add=True)


def _emit_out(s, acc, b_hbm, out_hbm, bv, stage):
    pltpu.sync_copy(b_hbm, bv)
    bvals = [bv[pl.ds(j * L, L)] for j in range(D // L)]

    for i in range(RPASS):
        cid = s + i * NS

        @pl.when(cid < NRCHUNK)
        def _():
            base = cid * RCHUNK
            pltpu.sync_copy(acc.at[pl.ds(base, RCHUNK)],
                            stage.at[pl.ds(0, RCHUNK)])

            def r_body(r, _):
                for j in range(D // L):
                    sl = pl.ds(j * L, L)
                    stage[r, sl] = stage[r, sl] + bvals[j]
                return 0

            lax.fori_loop(0, RCHUNK, r_body, 0)
            pltpu.sync_copy(stage.at[pl.ds(0, RCHUNK)],
                            out_hbm.at[pl.ds(base, RCHUNK)])


def _sc_body(h1, idx1, w1, h2, idx2, w2, b1, b2,
             x1_out, x2_out,
             acc, i0, i1, i2, i3, i4, i5,
             w0, w1b, w2b, w3b, w4b, w5b,
             r0, r1, bv,
             is0, is1, is2, is3, is4, is5,
             sem0, sem1):
    c = lax.axis_index("c")
    s = lax.axis_index("s")
    ibufs = [i0, i1, i2, i3, i4, i5]
    wbufs = [w0, w1b, w2b, w3b, w4b, w5b]
    isems = [is0, is1, is2, is3, is4, is5]
    rows = [r0, r1]
    sems = [sem0, sem1]
    stage = r0

    def z_body(r, _):
        for j in range(D // L):
            stage[r, pl.ds(j * L, L)] = jnp.zeros((L,), jnp.float32)
        return 0

    lax.fori_loop(0, RCHUNK, z_body, 0)
    for i in range(RPASS):
        cid = s + i * NS

        @pl.when(cid < NRCHUNK)
        def _():
            pltpu.sync_copy(stage.at[pl.ds(0, RCHUNK)],
                            acc.at[pl.ds(cid * RCHUNK, RCHUNK)])

    plsc.subcore_barrier()

    @pl.when(c == 0)
    def _():
        _process_branch(s, h1, idx1, w1, acc,
                        ibufs, wbufs, rows, isems, sems)

    @pl.when(c == 1)
    def _():
        _process_branch(s, h2, idx2, w2, acc,
                        ibufs, wbufs, rows, isems, sems)

    plsc.subcore_barrier()

    @pl.when(c == 0)
    def _():
        _emit_out(s, acc, b1, x1_out, bv, stage)

    @pl.when(c == 1)
    def _():
        _emit_out(s, acc, b2, x2_out, bv, stage)


@jax.jit
def _sc_scatter(h1, idx1, w1, h2, idx2, w2, b1, b2):
    mesh = plsc.VectorSubcoreMesh(core_axis_name="c", subcore_axis_name="s")
    return pl.kernel(
        _sc_body,
        out_type=[jax.ShapeDtypeStruct((N, D), jnp.float32)] * 2,
        mesh=mesh,
        scratch_types=(
            [pltpu.VMEM_SHARED((N, D), jnp.float32)]
            + [pltpu.VMEM((2, K), jnp.int32)] * NIB
            + [pltpu.VMEM((K,), jnp.float32)] * NIB
            + [pltpu.VMEM((K, D), jnp.float32)] * NBUF
            + [pltpu.VMEM((D,), jnp.float32)]
            + [pltpu.SemaphoreType.DMA] * (NIB + NBUF)
        ),
    )(h1, idx1, w1, h2, idx2, w2, b1, b2)


def kernel(x, edge_index, edge_weight, edge_index2, edge_weight2,
           W_ln, b_ln, W1, b1, W2, b2):
    x0, h1, h2 = _matmuls(x, W_ln, b_ln, W1, W2)
    x1, x2 = _sc_scatter(
        h1, edge_index.reshape(2 * E), edge_weight,
        h2, edge_index2.reshape(2 * E), edge_weight2,
        b1, b2)
    return (x0, x1, x2)


# speedup vs baseline: 1.4147x; 1.2897x over previous
import jax
import jax.numpy as jnp
from jax import lax
from jax.experimental import pallas as pl
from jax.experimental.pallas import tpu as pltpu
from jax.experimental.pallas import tpu_sc as plsc

N = 10000
E = 320000
D = 128
L = 16
NS = 16
K = 128
TOTAL = E // K
NITER = TOTAL // NS
NBUF = 2
NIB = 6
UNROLL = 6
RCHUNK = 80
NRCHUNK = N // RCHUNK
RPASS = (NRCHUNK + NS - 1) // NS

MM_BLOCK = 400


def _mm_body(x_ref, wln_ref, bln_ref, w1_ref, w2_ref, x0_ref, h1_ref, h2_ref):
    xb = x_ref[...]
    x0_ref[...] = (
        jnp.dot(xb, wln_ref[...], preferred_element_type=jnp.float32)
        + bln_ref[...]
    )
    h1_ref[...] = jnp.dot(xb, w1_ref[...], preferred_element_type=jnp.float32)
    h2_ref[...] = jnp.dot(xb, w2_ref[...], preferred_element_type=jnp.float32)


@jax.jit
def _matmuls(x, W_ln, b_ln, W1, W2):
    grid = (N // MM_BLOCK,)
    blk = pl.BlockSpec((MM_BLOCK, D), lambda i: (i, 0))
    wspec = pl.BlockSpec((D, D), lambda i: (0, 0))
    bspec = pl.BlockSpec((1, D), lambda i: (0, 0))
    return pl.pallas_call(
        _mm_body,
        grid=grid,
        in_specs=[blk, wspec, bspec, wspec, wspec],
        out_specs=[blk, blk, blk],
        out_shape=[jax.ShapeDtypeStruct((N, D), jnp.float32)] * 3,
    )(x, W_ln, b_ln.reshape(1, D), W1, W2)


def _process_branch(s, h_hbm, idx_hbm, w_hbm, acc,
                    ibufs, wbufs, rows, isems, sems):
    def cofs(i):
        return (s + NS * i) * K

    def istart(i, u):
        o = cofs(i)
        pltpu.async_copy(idx_hbm.at[pl.ds(o, K)], ibufs[u].at[0], isems[u])
        pltpu.async_copy(idx_hbm.at[pl.ds(E + o, K)], ibufs[u].at[1],
                         isems[u])
        pltpu.async_copy(w_hbm.at[pl.ds(o, K)], wbufs[u], isems[u])

    def iwait(i, u):
        o = cofs(i)
        pltpu.make_async_copy(
            idx_hbm.at[pl.ds(o, K)], ibufs[u].at[0], isems[u]).wait()
        pltpu.make_async_copy(
            idx_hbm.at[pl.ds(E + o, K)], ibufs[u].at[1], isems[u]).wait()
        pltpu.make_async_copy(
            w_hbm.at[pl.ds(o, K)], wbufs[u], isems[u]).wait()

    for i in range(NBUF + 2):
        istart(i, i)

    iwait(0, 0)
    pltpu.async_copy(h_hbm.at[ibufs[0].at[0]], rows[0], sems[0])

    def mult(wb, r):
        def grp_body(g16, _):
            w16 = wb[pl.ds(g16 * L, L)]
            for i in range(L):
                e = g16 * L + i
                ws = w16[i]
                for j in range(D // L):
                    sl = pl.ds(j * L, L)
                    r[e, sl] = r[e, sl] * ws
            return 0

        lax.fori_loop(0, K // L, grp_body, 0)

    def outer(gg, _):
        for u in range(UNROLL):
            i = gg * UNROLL + u
            b = u % NBUF
            bp = (u - 1) % NBUF
            ib = u % NIB
            ibp = (u - 1) % NIB
            ibn = (u + 1) % NIB
            ib4 = (u + 4) % NIB

            pltpu.make_async_copy(
                h_hbm.at[ibufs[ib].at[0]], rows[b], sems[b]).wait()

            @pl.when(i > 0)
            def _():
                pltpu.make_async_copy(
                    rows[bp], acc.at[ibufs[ibp].at[1]], sems[bp]).wait()

            @pl.when(s + NS * (i + 1) < TOTAL)
            def _():
                iwait(i + 1, ibn)
                pltpu.async_copy(
                    h_hbm.at[ibufs[ibn].at[0]], rows[bp], sems[bp])

            mult(wbufs[ib], rows[b])

            @pl.when(s + NS * (i + 4) < TOTAL)
            def _():
                istart(i + 4, ib4)

            pltpu.async_copy(
                rows[b], acc.at[ibufs[ib].at[1]], sems[b], add=True)
        return 0

    lax.fori_loop(0, NITER // UNROLL, outer, 0)

    ul = (NITER - 1) % UNROLL
    pltpu.make_async_copy(
        rows[ul % NBUF], acc.at[ibufs[ul % NIB].at[1]],
        sems[ul % NBUF]).wait()

    @pl.when(s + NS * NITER < TOTAL)
    def _():
        ut = NITER % UNROLL
        pltpu.make_async_copy(
            h_hbm.at[ibufs[ut % NIB].at[0]], rows[ut % NBUF],
            sems[ut % NBUF]).wait()
        mult(wbufs[ut % NIB], rows[ut % NBUF])
        pltpu.sync_copy(rows[ut % NBUF], acc.at[ibufs[ut % NIB].at[1]],
                        add=True)


def _emit_out(s, acc, b_hbm, out_hbm, bv, stage):
    pltpu.sync_copy(b_hbm, bv)
    bvals = [bv[pl.ds(j * L, L)] for j in range(D // L)]

    for i in range(RPASS):
        cid = s + i * NS

        @pl.when(cid < NRCHUNK)
        def _():
            base = cid * RCHUNK
            pltpu.sync_copy(acc.at[pl.ds(base, RCHUNK)],
                            stage.at[pl.ds(0, RCHUNK)])

            def r_body(r, _):
                for j in range(D // L):
                    sl = pl.ds(j * L, L)
                    stage[r, sl] = stage[r, sl] + bvals[j]
                return 0

            lax.fori_loop(0, RCHUNK, r_body, 0)
            pltpu.sync_copy(stage.at[pl.ds(0, RCHUNK)],
                            out_hbm.at[pl.ds(base, RCHUNK)])


def _sc_body(h1, idx1, w1, h2, idx2, w2, b1, b2,
             x1_out, x2_out,
             acc, i0, i1, i2, i3, i4, i5,
             w0, w1b, w2b, w3b, w4b, w5b,
             r0, r1, bv,
             is0, is1, is2, is3, is4, is5,
             sem0, sem1):
    c = lax.axis_index("c")
    s = lax.axis_index("s")
    ibufs = [i0, i1, i2, i3, i4, i5]
    wbufs = [w0, w1b, w2b, w3b, w4b, w5b]
    isems = [is0, is1, is2, is3, is4, is5]
    rows = [r0, r1]
    sems = [sem0, sem1]
    stage = r0

    def z_body(r, _):
        for j in range(D // L):
            stage[r, pl.ds(j * L, L)] = jnp.zeros((L,), jnp.float32)
        return 0

    lax.fori_loop(0, RCHUNK, z_body, 0)
    for i in range(RPASS):
        cid = s + i * NS

        @pl.when(cid < NRCHUNK)
        def _():
            pltpu.sync_copy(stage.at[pl.ds(0, RCHUNK)],
                            acc.at[pl.ds(cid * RCHUNK, RCHUNK)])

    plsc.subcore_barrier()

    @pl.when(c == 0)
    def _():
        _process_branch(s, h1, idx1, w1, acc,
                        ibufs, wbufs, rows, isems, sems)

    @pl.when(c == 1)
    def _():
        _process_branch(s, h2, idx2, w2, acc,
                        ibufs, wbufs, rows, isems, sems)

    plsc.subcore_barrier()

    @pl.when(c == 0)
    def _():
        _emit_out(s, acc, b1, x1_out, bv, stage)

    @pl.when(c == 1)
    def _():
        _emit_out(s, acc, b2, x2_out, bv, stage)


@jax.jit
def _sc_scatter(h1, idx1, w1, h2, idx2, w2, b1, b2):
    mesh = plsc.VectorSubcoreMesh(core_axis_name="c", subcore_axis_name="s")
    return pl.kernel(
        _sc_body,
        out_type=[jax.ShapeDtypeStruct((N, D), jnp.float32)] * 2,
        mesh=mesh,
        scratch_types=(
            [pltpu.VMEM_SHARED((N, D), jnp.float32)]
            + [pltpu.VMEM((2, K), jnp.int32)] * NIB
            + [pltpu.VMEM((K,), jnp.float32)] * NIB
            + [pltpu.VMEM((K, D), jnp.float32)] * NBUF
            + [pltpu.VMEM((D,), jnp.float32)]
            + [pltpu.SemaphoreType.DMA] * (NIB + NBUF)
        ),
    )(h1, idx1, w1, h2, idx2, w2, b1, b2)


def kernel(x, edge_index, edge_weight, edge_index2, edge_weight2,
           W_ln, b_ln, W1, b1, W2, b2):
    x0, h1, h2 = _matmuls(x, W_ln, b_ln, W1, W2)
    x1, x2 = _sc_scatter(
        h1, edge_index.reshape(2 * E), edge_weight,
        h2, edge_index2.reshape(2 * E), edge_weight2,
        b1, b2)
    return (x0, x1, x2)


# speedup vs baseline: 1.5106x; 1.0678x over previous
import jax
import jax.numpy as jnp
from jax import lax
from jax.experimental import pallas as pl
from jax.experimental.pallas import tpu as pltpu
from jax.experimental.pallas import tpu_sc as plsc

N = 10000
E = 320000
D = 128
L = 16
NS = 16
K = 128
TOTAL = E // K
NITER = TOTAL // NS
NBUF = 2
NIB = 6
UNROLL = 6
RCHUNK = 80
NRCHUNK = N // RCHUNK
RPASS = (NRCHUNK + NS - 1) // NS

MM_BLOCK = 2000


def _mm_body(x_ref, wln_ref, bln_ref, w1_ref, w2_ref, x0_ref, h1_ref, h2_ref):
    xb = x_ref[...]
    x0_ref[...] = (
        jnp.dot(xb, wln_ref[...], preferred_element_type=jnp.float32)
        + bln_ref[...]
    )
    h1_ref[...] = jnp.dot(xb, w1_ref[...], preferred_element_type=jnp.float32)
    h2_ref[...] = jnp.dot(xb, w2_ref[...], preferred_element_type=jnp.float32)


@jax.jit
def _matmuls(x, W_ln, b_ln, W1, W2):
    grid = (N // MM_BLOCK,)
    blk = pl.BlockSpec((MM_BLOCK, D), lambda i: (i, 0))
    wspec = pl.BlockSpec((D, D), lambda i: (0, 0))
    bspec = pl.BlockSpec((1, D), lambda i: (0, 0))
    return pl.pallas_call(
        _mm_body,
        grid=grid,
        in_specs=[blk, wspec, bspec, wspec, wspec],
        out_specs=[blk, blk, blk],
        out_shape=[jax.ShapeDtypeStruct((N, D), jnp.float32)] * 3,
    )(x, W_ln, b_ln.reshape(1, D), W1, W2)


def _process_branch(s, h_hbm, idx_hbm, w_hbm, acc,
                    ibufs, wbufs, rows, isems, sems):
    def cofs(i):
        return (s + NS * i) * K

    def istart(i, u):
        o = cofs(i)
        pltpu.async_copy(idx_hbm.at[0, pl.ds(o, K)], ibufs[u].at[0], isems[u])
        pltpu.async_copy(idx_hbm.at[1, pl.ds(o, K)], ibufs[u].at[1],
                         isems[u])
        pltpu.async_copy(w_hbm.at[pl.ds(o, K)], wbufs[u], isems[u])

    def iwait(i, u):
        o = cofs(i)
        pltpu.make_async_copy(
            idx_hbm.at[0, pl.ds(o, K)], ibufs[u].at[0], isems[u]).wait()
        pltpu.make_async_copy(
            idx_hbm.at[1, pl.ds(o, K)], ibufs[u].at[1], isems[u]).wait()
        pltpu.make_async_copy(
            w_hbm.at[pl.ds(o, K)], wbufs[u], isems[u]).wait()

    for i in range(NBUF + 2):
        istart(i, i)

    iwait(0, 0)
    pltpu.async_copy(h_hbm.at[ibufs[0].at[0]], rows[0], sems[0])

    def mult(wb, r):
        def grp_body(g16, _):
            w16 = wb[pl.ds(g16 * L, L)]
            for i in range(L):
                e = g16 * L + i
                ws = w16[i]
                for j in range(D // L):
                    sl = pl.ds(j * L, L)
                    r[e, sl] = r[e, sl] * ws
            return 0

        lax.fori_loop(0, K // L, grp_body, 0)

    def outer(gg, _):
        for u in range(UNROLL):
            i = gg * UNROLL + u
            b = u % NBUF
            bp = (u - 1) % NBUF
            ib = u % NIB
            ibp = (u - 1) % NIB
            ibn = (u + 1) % NIB
            ib4 = (u + 4) % NIB

            pltpu.make_async_copy(
                h_hbm.at[ibufs[ib].at[0]], rows[b], sems[b]).wait()

            @pl.when(i > 0)
            def _():
                pltpu.make_async_copy(
                    rows[bp], acc.at[ibufs[ibp].at[1]], sems[bp]).wait()

            @pl.when(s + NS * (i + 1) < TOTAL)
            def _():
                iwait(i + 1, ibn)
                pltpu.async_copy(
                    h_hbm.at[ibufs[ibn].at[0]], rows[bp], sems[bp])

            mult(wbufs[ib], rows[b])

            @pl.when(s + NS * (i + 4) < TOTAL)
            def _():
                istart(i + 4, ib4)

            pltpu.async_copy(
                rows[b], acc.at[ibufs[ib].at[1]], sems[b], add=True)
        return 0

    lax.fori_loop(0, NITER // UNROLL, outer, 0)

    ul = (NITER - 1) % UNROLL
    pltpu.make_async_copy(
        rows[ul % NBUF], acc.at[ibufs[ul % NIB].at[1]],
        sems[ul % NBUF]).wait()

    @pl.when(s + NS * NITER < TOTAL)
    def _():
        ut = NITER % UNROLL
        pltpu.make_async_copy(
            h_hbm.at[ibufs[ut % NIB].at[0]], rows[ut % NBUF],
            sems[ut % NBUF]).wait()
        mult(wbufs[ut % NIB], rows[ut % NBUF])
        pltpu.sync_copy(rows[ut % NBUF], acc.at[ibufs[ut % NIB].at[1]],
                        add=True)


def _emit_out(s, acc, b_hbm, out_hbm, bv, stage):
    pltpu.sync_copy(b_hbm, bv)
    bvals = [bv[pl.ds(j * L, L)] for j in range(D // L)]

    for i in range(RPASS):
        cid = s + i * NS

        @pl.when(cid < NRCHUNK)
        def _():
            base = cid * RCHUNK
            pltpu.sync_copy(acc.at[pl.ds(base, RCHUNK)],
                            stage.at[pl.ds(0, RCHUNK)])

            def r_body(r, _):
                for j in range(D // L):
                    sl = pl.ds(j * L, L)
                    stage[r, sl] = stage[r, sl] + bvals[j]
                return 0

            lax.fori_loop(0, RCHUNK, r_body, 0)
            pltpu.sync_copy(stage.at[pl.ds(0, RCHUNK)],
                            out_hbm.at[pl.ds(base, RCHUNK)])


def _sc_body(h1, idx1, w1, h2, idx2, w2, b1, b2,
             x1_out, x2_out,
             acc, i0, i1, i2, i3, i4, i5,
             w0, w1b, w2b, w3b, w4b, w5b,
             r0, r1, bv,
             is0, is1, is2, is3, is4, is5,
             sem0, sem1):
    c = lax.axis_index("c")
    s = lax.axis_index("s")
    ibufs = [i0, i1, i2, i3, i4, i5]
    wbufs = [w0, w1b, w2b, w3b, w4b, w5b]
    isems = [is0, is1, is2, is3, is4, is5]
    rows = [r0, r1]
    sems = [sem0, sem1]
    stage = r0

    def z_body(r, _):
        for j in range(D // L):
            stage[r, pl.ds(j * L, L)] = jnp.zeros((L,), jnp.float32)
        return 0

    lax.fori_loop(0, RCHUNK, z_body, 0)
    for i in range(RPASS):
        cid = s + i * NS

        @pl.when(cid < NRCHUNK)
        def _():
            pltpu.sync_copy(stage.at[pl.ds(0, RCHUNK)],
                            acc.at[pl.ds(cid * RCHUNK, RCHUNK)])

    plsc.subcore_barrier()

    @pl.when(c == 0)
    def _():
        _process_branch(s, h1, idx1, w1, acc,
                        ibufs, wbufs, rows, isems, sems)

    @pl.when(c == 1)
    def _():
        _process_branch(s, h2, idx2, w2, acc,
                        ibufs, wbufs, rows, isems, sems)

    plsc.subcore_barrier()

    @pl.when(c == 0)
    def _():
        _emit_out(s, acc, b1, x1_out, bv, stage)

    @pl.when(c == 1)
    def _():
        _emit_out(s, acc, b2, x2_out, bv, stage)


@jax.jit
def _sc_scatter(h1, idx1, w1, h2, idx2, w2, b1, b2):
    mesh = plsc.VectorSubcoreMesh(core_axis_name="c", subcore_axis_name="s")
    return pl.kernel(
        _sc_body,
        out_type=[jax.ShapeDtypeStruct((N, D), jnp.float32)] * 2,
        mesh=mesh,
        scratch_types=(
            [pltpu.VMEM_SHARED((N, D), jnp.float32)]
            + [pltpu.VMEM((2, K), jnp.int32)] * NIB
            + [pltpu.VMEM((K,), jnp.float32)] * NIB
            + [pltpu.VMEM((K, D), jnp.float32)] * NBUF
            + [pltpu.VMEM((D,), jnp.float32)]
            + [pltpu.SemaphoreType.DMA] * (NIB + NBUF)
        ),
    )(h1, idx1, w1, h2, idx2, w2, b1, b2)


def kernel(x, edge_index, edge_weight, edge_index2, edge_weight2,
           W_ln, b_ln, W1, b1, W2, b2):
    x0, h1, h2 = _matmuls(x, W_ln, b_ln, W1, W2)
    x1, x2 = _sc_scatter(
        h1, edge_index, edge_weight,
        h2, edge_index2, edge_weight2,
        b1, b2)
    return (x0, x1, x2)


# speedup vs baseline: 1.6206x; 1.0728x over previous
import jax
import jax.numpy as jnp
from jax import lax
from jax.experimental import pallas as pl
from jax.experimental.pallas import tpu as pltpu
from jax.experimental.pallas import tpu_sc as plsc

N = 10000
E = 320000
D = 128
L = 16
NS = 16
K = 128
TOTAL = E // K
NITER = TOTAL // NS
NBUF = 3
RCHUNK = 80
NRCHUNK = N // RCHUNK
RPASS = (NRCHUNK + NS - 1) // NS

MM_BLOCK = 2000


def _mm_body(x_ref, wln_ref, bln_ref, w1_ref, w2_ref, x0_ref, h1_ref, h2_ref):
    xb = x_ref[...]
    x0_ref[...] = (
        jnp.dot(xb, wln_ref[...], preferred_element_type=jnp.float32)
        + bln_ref[...]
    )
    h1_ref[...] = jnp.dot(xb, w1_ref[...], preferred_element_type=jnp.float32)
    h2_ref[...] = jnp.dot(xb, w2_ref[...], preferred_element_type=jnp.float32)


@jax.jit
def _matmuls(x, W_ln, b_ln, W1, W2):
    grid = (N // MM_BLOCK,)
    blk = pl.BlockSpec((MM_BLOCK, D), lambda i: (i, 0))
    wspec = pl.BlockSpec((D, D), lambda i: (0, 0))
    bspec = pl.BlockSpec((1, D), lambda i: (0, 0))
    return pl.pallas_call(
        _mm_body,
        grid=grid,
        in_specs=[blk, wspec, bspec, wspec, wspec],
        out_specs=[blk, blk, blk],
        out_shape=[jax.ShapeDtypeStruct((N, D), jnp.float32)] * 3,
    )(x, W_ln, b_ln.reshape(1, D), W1, W2)


def _process_branch(s, h_hbm, idx_hbm, w_hbm, acc,
                    srcv, dstv, wv, rows, isems, sems):
    def cofs(i):
        return (s + NS * i) * K

    def istart(i, u):
        o = cofs(i)
        pltpu.async_copy(idx_hbm.at[0, pl.ds(o, K)],
                         srcv.at[pl.ds(u * K, K)], isems[u])
        pltpu.async_copy(idx_hbm.at[1, pl.ds(o, K)], dstv.at[u], isems[u])
        pltpu.async_copy(w_hbm.at[pl.ds(o, K)],
                         wv.at[pl.ds(u * K, K)], isems[u])

    def iwait(i, u):
        o = cofs(i)
        pltpu.make_async_copy(
            idx_hbm.at[0, pl.ds(o, K)],
            srcv.at[pl.ds(u * K, K)], isems[u]).wait()
        pltpu.make_async_copy(
            idx_hbm.at[1, pl.ds(o, K)], dstv.at[u], isems[u]).wait()
        pltpu.make_async_copy(
            w_hbm.at[pl.ds(o, K)], wv.at[pl.ds(u * K, K)], isems[u]).wait()

    istart(0, 0)
    istart(1, 1)
    iwait(0, 0)
    pltpu.async_copy(h_hbm.at[srcv.at[pl.ds(0, K)]], rows[0], sems[0])

    def mult(u, r):
        def grp_body(g16, _):
            w16 = wv[pl.ds(u * K + g16 * L, L)]
            for i in range(L):
                e = g16 * L + i
                ws = w16[i]
                for j in range(D // L):
                    sl = pl.ds(j * L, L)
                    r[e, sl] = r[e, sl] * ws
            return 0

        lax.fori_loop(0, K // L, grp_body, 0)

    def outer(gg, _):
        for u in range(NBUF):
            i = gg * NBUF + u
            up = (u - 1) % NBUF
            un = (u + 1) % NBUF

            @pl.when(s + NS * (i + 1) < TOTAL)
            def _():
                iwait(i + 1, un)
                pltpu.async_copy(
                    h_hbm.at[srcv.at[pl.ds(un * K, K)]], rows[un], sems[un])

            pltpu.make_async_copy(
                h_hbm.at[srcv.at[pl.ds(u * K, K)]], rows[u], sems[u]).wait()

            @pl.when(i > 0)
            def _():
                pltpu.make_async_copy(
                    rows[up], acc.at[dstv.at[up]], sems[up]).wait()

            @pl.when(s + NS * (i + 2) < TOTAL)
            def _():
                istart(i + 2, up)

            mult(u, rows[u])

            pltpu.async_copy(rows[u], acc.at[dstv.at[u]], sems[u], add=True)
        return 0

    lax.fori_loop(0, NITER // NBUF, outer, 0)

    ul = (NITER - 1) % NBUF
    pltpu.make_async_copy(
        rows[ul], acc.at[dstv.at[ul]], sems[ul]).wait()

    @pl.when(s + NS * NITER < TOTAL)
    def _():
        ut = NITER % NBUF
        pltpu.make_async_copy(
            h_hbm.at[srcv.at[pl.ds(ut * K, K)]], rows[ut], sems[ut]).wait()
        mult(ut, rows[ut])
        pltpu.sync_copy(rows[ut], acc.at[dstv.at[ut]], add=True)


def _emit_out(s, acc, b_hbm, out_hbm, wv, stage):
    pltpu.sync_copy(b_hbm, wv.at[pl.ds(0, D)])
    bvals = [wv[pl.ds(j * L, L)] for j in range(D // L)]

    for i in range(RPASS):
        cid = s + i * NS

        @pl.when(cid < NRCHUNK)
        def _():
            base = cid * RCHUNK
            pltpu.sync_copy(acc.at[pl.ds(base, RCHUNK)],
                            stage.at[pl.ds(0, RCHUNK)])

            def r_body(r, _):
                for j in range(D // L):
                    sl = pl.ds(j * L, L)
                    stage[r, sl] = stage[r, sl] + bvals[j]
                return 0

            lax.fori_loop(0, RCHUNK, r_body, 0)
            pltpu.sync_copy(stage.at[pl.ds(0, RCHUNK)],
                            out_hbm.at[pl.ds(base, RCHUNK)])


def _sc_body(h1, idx1, w1, h2, idx2, w2, b1, b2,
             x1_out, x2_out,
             acc, srcv, dstv, wv,
             r0, r1, r2,
             is0, is1, is2,
             sem0, sem1, sem2):
    c = lax.axis_index("c")
    s = lax.axis_index("s")
    isems = [is0, is1, is2]
    rows = [r0, r1, r2]
    sems = [sem0, sem1, sem2]
    stage = r0

    def z_body(r, _):
        for j in range(D // L):
            stage[r, pl.ds(j * L, L)] = jnp.zeros((L,), jnp.float32)
        return 0

    lax.fori_loop(0, RCHUNK, z_body, 0)
    for i in range(RPASS):
        cid = s + i * NS

        @pl.when(cid < NRCHUNK)
        def _():
            pltpu.sync_copy(stage.at[pl.ds(0, RCHUNK)],
                            acc.at[pl.ds(cid * RCHUNK, RCHUNK)])

    plsc.subcore_barrier()

    @pl.when(c == 0)
    def _():
        _process_branch(s, h1, idx1, w1, acc,
                        srcv, dstv, wv, rows, isems, sems)

    @pl.when(c == 1)
    def _():
        _process_branch(s, h2, idx2, w2, acc,
                        srcv, dstv, wv, rows, isems, sems)

    plsc.subcore_barrier()

    @pl.when(c == 0)
    def _():
        _emit_out(s, acc, b1, x1_out, wv, stage)

    @pl.when(c == 1)
    def _():
        _emit_out(s, acc, b2, x2_out, wv, stage)


@jax.jit
def _sc_scatter(h1, idx1, w1, h2, idx2, w2, b1, b2):
    mesh = plsc.VectorSubcoreMesh(core_axis_name="c", subcore_axis_name="s")
    return pl.kernel(
        _sc_body,
        out_type=[jax.ShapeDtypeStruct((N, D), jnp.float32)] * 2,
        mesh=mesh,
        scratch_types=(
            [pltpu.VMEM_SHARED((N, D), jnp.float32)]
            + [pltpu.VMEM((NBUF * K,), jnp.int32)]
            + [pltpu.VMEM((NBUF, K), jnp.int32)]
            + [pltpu.VMEM((NBUF * K,), jnp.float32)]
            + [pltpu.VMEM((K, D), jnp.float32)] * NBUF
            + [pltpu.SemaphoreType.DMA] * (2 * NBUF)
        ),
    )(h1, idx1, w1, h2, idx2, w2, b1, b2)


def kernel(x, edge_index, edge_weight, edge_index2, edge_weight2,
           W_ln, b_ln, W1, b1, W2, b2):
    x0, h1, h2 = _matmuls(x, W_ln, b_ln, W1, W2)
    x1, x2 = _sc_scatter(
        h1, edge_index, edge_weight,
        h2, edge_index2, edge_weight2,
        b1, b2)
    return (x0, x1, x2)
